# baseline (speedup 1.0000x reference)
import functools

import jax
import jax.numpy as jnp
from jax import lax
from jax.experimental import pallas as pl
from jax.experimental.pallas import tpu as pltpu
from jax.experimental.pallas import tpu_sc as plsc

B, N, D = 4, 8192, 2048
DS = 1024
DT = D - DS

NC, NS, L = 2, 16, 16
NW = NC * NS
COLS_PER_W = (B * DS) // NW
CW = COLS_PER_W
G = CW // L
RB = 256
NRC = N // RB
UNROLL = 4

_mesh = plsc.VectorSubcoreMesh(core_axis_name="c", subcore_axis_name="s")


@functools.partial(
    pl.kernel,
    out_type=jax.ShapeDtypeStruct((B * DS,), jnp.int32),
    mesh=_mesh,
    scratch_types=[
        pltpu.VMEM((RB, CW), jnp.float32),
        pltpu.VMEM((RB, CW), jnp.float32),
        pltpu.VMEM((COLS_PER_W,), jnp.int32),
        pltpu.SemaphoreType.DMA,
        pltpu.SemaphoreType.DMA,
    ],
)
def _argmin_sc(x_hbm, out_hbm, buf0, buf1, outv, sem0, sem1):
    wid = lax.axis_index("s") * NC + lax.axis_index("c")
    base = wid * COLS_PER_W
    b = base // DS
    j0 = base % DS

    bufs = (buf0, buf1)
    sems = (sem0, sem1)

    def copy(rc, ph):
        return pltpu.make_async_copy(
            x_hbm.at[b, pl.ds(rc * RB, RB), pl.ds(j0, CW)],
            bufs[ph], sems[ph])

    def compute(buf, r0, carry):
        def row_body(r, carry2):
            mv, mi = carry2
            rv = jnp.full((L,), r0 + r, jnp.int32)
            mv2, mi2 = [], []
            for g in range(G):
                v = buf[r, g * L:(g + 1) * L]
                p = v < mv[g]
                mv2.append(jnp.minimum(v, mv[g]))
                mi2.append(jnp.where(p, rv, mi[g]))
            return (tuple(mv2), tuple(mi2))

        return lax.fori_loop(0, RB, row_body, carry, unroll=UNROLL)

    copy(0, 0).start()

    def pair_body(i, carry):
        rc0 = 2 * i
        copy(rc0 + 1, 1).start()
        copy(rc0, 0).wait()
        carry = compute(buf0, rc0 * RB, carry)

        @pl.when(rc0 + 2 < NRC)
        def _():
            copy(rc0 + 2, 0).start()

        copy(rc0 + 1, 1).wait()
        carry = compute(buf1, (rc0 + 1) * RB, carry)
        return carry

    init = (
        tuple(jnp.full((L,), jnp.inf, jnp.float32) for _ in range(G)),
        tuple(jnp.zeros((L,), jnp.int32) for _ in range(G)),
    )
    _, minis = lax.fori_loop(0, NRC // 2, pair_body, init)
    for g in range(G):
        outv[g * L:(g + 1) * L] = minis[g]

    pltpu.sync_copy(outv, out_hbm.at[pl.ds(base, COLS_PER_W)])


BJ = 512
RC = 512
NRC_TC = N // RC


def _argmin_tc_body(x_ref, o_ref, mv_ref, mi_ref):
    i = pl.program_id(2)

    @pl.when(i == 0)
    def _():
        mv_ref[...] = jnp.full((1, BJ), jnp.inf, jnp.float32)
        mi_ref[...] = jnp.zeros((1, BJ), jnp.int32)

    xb = x_ref[0]
    cm = jnp.min(xb, axis=0, keepdims=True)
    iota = lax.broadcasted_iota(jnp.int32, (RC, BJ), 0)
    ci = jnp.min(jnp.where(xb == cm, iota, N), axis=0, keepdims=True)
    ci = ci + i * RC
    mv = mv_ref[...]
    p = cm < mv
    mv_ref[...] = jnp.minimum(cm, mv)
    mi_ref[...] = jnp.where(p, ci, mi_ref[...])

    @pl.when(i == NRC_TC - 1)
    def _():
        o_ref[...] = mi_ref[...].reshape(1, 1, BJ)


_argmin_tc = pl.pallas_call(
    _argmin_tc_body,
    grid=(B, DT // BJ, NRC_TC),
    in_specs=[pl.BlockSpec((1, RC, BJ), lambda b, j, i: (b, i, (DS // BJ) + j))],
    out_specs=pl.BlockSpec((1, 1, BJ), lambda b, j, i: (b, 0, j)),
    out_shape=jax.ShapeDtypeStruct((B, 1, DT), jnp.int32),
    scratch_shapes=[
        pltpu.VMEM((1, BJ), jnp.float32),
        pltpu.VMEM((1, BJ), jnp.int32),
    ],
)


def kernel(x):
    sc_out = _argmin_sc(x).reshape(B, DS)
    tc_out = _argmin_tc(x).reshape(B, DT)
    out = jnp.concatenate([sc_out, tc_out], axis=1)
    return out.astype(jnp.int64)
